# baseline (speedup 1.0000x reference)
import functools

import jax
import jax.numpy as jnp
from jax import lax
from jax.experimental import pallas as pl
from jax.experimental.pallas import tpu as pltpu
from jax.experimental.pallas import tpu_sc as plsc

B = 1024
L = 512
D = 768
NC = 2
NS = 16
NW = NC * NS
N_TOK = B * L
TOK_PER_W = N_TOK // NW
C = 32
N_CHUNK = TOK_PER_W // C
BLK = 1024
CPB = BLK // C
DSL = D // 16


def _sc_body(x_hbm, seg_hbm, word_hbm, combo_hbm, out_hbm,
             rows0, rows1, add0, add1, cid0, cid1,
             xb0, xb1, sb,
             gw0, gw1, gc0, gc1, wr0, wr1):
    wid = lax.axis_index("s") * NC + lax.axis_index("c")
    base = wid * TOK_PER_W
    iota = lax.iota(jnp.int32, 16)

    bufs = ((rows0, add0, cid0, gw0, gc0, wr0),
            (rows1, add1, cid1, gw1, gc1, wr1))
    xblks = (xb0, xb1)

    def issue_combo(c, buf):
        _, addv, cidx, _, gc, _ = buf
        blk = lax.div(c, CPB)
        off = lax.rem(c, CPB) * C
        for par in range(2):
            @pl.when((lax.rem(blk, 2) == par) & (off == 0))
            def _():
                pltpu.sync_copy(x_hbm.at[pl.ds(base + blk * BLK, BLK)],
                                xblks[par])
                pltpu.sync_copy(seg_hbm.at[pl.ds(base + blk * BLK, BLK)], sb)

        p0 = lax.rem(c * C, L)
        for u in range(C // 16):
            s16 = sb[pl.ds(off + u * 16, 16)]
            cidx[pl.ds(u * 16, 16)] = s16 * L + (iota + u * 16) + p0
        pltpu.async_copy(combo_hbm.at[cidx], addv, gc)

    def issue_word(c, buf):
        rows, _, _, gw, _, _ = buf
        blk = lax.div(c, CPB)
        off = lax.rem(c, CPB) * C
        for par in range(2):
            @pl.when(lax.rem(blk, 2) == par)
            def _():
                pltpu.async_copy(word_hbm.at[xblks[par].at[pl.ds(off, C)]],
                                 rows, gw)

    def wait_in(c, buf):
        rows, addv, cidx, gw, gc, _ = buf
        blk = lax.div(c, CPB)
        off = lax.rem(c, CPB) * C
        for par in range(2):
            @pl.when(lax.rem(blk, 2) == par)
            def _():
                pltpu.make_async_copy(
                    word_hbm.at[xblks[par].at[pl.ds(off, C)]], rows, gw).wait()
        pltpu.make_async_copy(combo_hbm.at[cidx], addv, gc).wait()

    def accumulate(buf):
        rows, addv = buf[0], buf[1]

        def per_row(ci, _):
            for j in range(DSL):
                plsc.addupdate(rows.at[ci, pl.ds(j * 16, 16)],
                               addv[ci, pl.ds(j * 16, 16)])
            return 0

        lax.fori_loop(0, C, per_row, 0, unroll=2)

    def issue_out(c, buf):
        rows, wr = buf[0], buf[5]
        pltpu.async_copy(rows, out_hbm.at[pl.ds(base + c * C, C)], wr)

    def wait_out(buf):
        rows, wr = buf[0], buf[5]
        pltpu.make_async_copy(rows, out_hbm.at[pl.ds(base, C)], wr).wait()

    issue_combo(0, bufs[0])
    issue_word(0, bufs[0])

    def pair(i, carry):
        c0 = 2 * i
        for p in range(2):
            c = c0 + p
            cur, nxt = bufs[p], bufs[1 - p]

            @pl.when(c + 1 < N_CHUNK)
            def _():
                issue_combo(c + 1, nxt)

            @pl.when(c >= 1)
            def _():
                wait_out(nxt)

            @pl.when(c + 1 < N_CHUNK)
            def _():
                issue_word(c + 1, nxt)

            wait_in(c, cur)
            accumulate(cur)
            issue_out(c, cur)
        return carry

    lax.fori_loop(0, N_CHUNK // 2, pair, 0)
    wait_out(bufs[1])


@functools.partial(jax.jit, static_argnames=())
def _sc_embed(x_flat, seg_flat, word_table, combo):
    mesh = plsc.VectorSubcoreMesh(core_axis_name="c", subcore_axis_name="s",
                                  num_cores=NC, num_subcores=NS)
    f = pl.kernel(
        _sc_body,
        out_type=jax.ShapeDtypeStruct((N_TOK, D), jnp.float32),
        mesh=mesh,
        scratch_types=[
            pltpu.VMEM((C, D), jnp.float32),
            pltpu.VMEM((C, D), jnp.float32),
            pltpu.VMEM((C, D), jnp.float32),
            pltpu.VMEM((C, D), jnp.float32),
            pltpu.VMEM((C,), jnp.int32),
            pltpu.VMEM((C,), jnp.int32),
            pltpu.VMEM((BLK,), jnp.int32),
            pltpu.VMEM((BLK,), jnp.int32),
            pltpu.VMEM((BLK,), jnp.int32),
            pltpu.SemaphoreType.DMA,
            pltpu.SemaphoreType.DMA,
            pltpu.SemaphoreType.DMA,
            pltpu.SemaphoreType.DMA,
            pltpu.SemaphoreType.DMA,
            pltpu.SemaphoreType.DMA,
        ],
    )
    return f(x_flat, seg_flat, word_table, combo)


def _mask_body(x_ref, o_ref):
    o_ref[...] = x_ref[...] > 0


def _mask(x):
    return pl.pallas_call(
        _mask_body,
        out_shape=jax.ShapeDtypeStruct((B, L), jnp.bool_),
        grid=(8,),
        in_specs=[pl.BlockSpec((B // 8, L), lambda i: (i, 0))],
        out_specs=pl.BlockSpec((B // 8, L), lambda i: (i, 0)),
    )(x)


def kernel(x, seg, word_table, pos_table, seg_table):
    combo = (seg_table[:, None, :] + pos_table[None, :, :]).reshape(2 * L, D)
    out_flat = _sc_embed(x.reshape(N_TOK), seg.reshape(N_TOK), word_table,
                         combo)
    return out_flat.reshape(B, L, D), _mask(x)


# speedup vs baseline: 1.2265x; 1.2265x over previous
import functools

import jax
import jax.numpy as jnp
from jax import lax
from jax.experimental import pallas as pl
from jax.experimental.pallas import tpu as pltpu
from jax.experimental.pallas import tpu_sc as plsc

B = 1024
L = 512
D = 768
NC = 2
NS = 16
NW = NC * NS
N_TOK = B * L
TOK_PER_W = N_TOK // NW
C = 32
N_CHUNK = TOK_PER_W // C
BLK = 1024
CPB = BLK // C
DSL = D // 16


def _sc_body(x_hbm, seg_hbm, word_hbm, combo_hbm, out_hbm,
             rows0, rows1, add0, add1, cid0, cid1,
             xb0, xb1, sb,
             gw0, gw1, gc0, gc1, wr0, wr1):
    wid = lax.axis_index("s") * NC + lax.axis_index("c")
    base = wid * TOK_PER_W
    iota = lax.iota(jnp.int32, 16)

    bufs = ((rows0, add0, cid0, gw0, gc0, wr0),
            (rows1, add1, cid1, gw1, gc1, wr1))
    xblks = (xb0, xb1)

    def issue_in(c, buf):
        rows, addv, cidx, gw, gc, _ = buf
        blk = lax.div(c, CPB)
        off = lax.rem(c, CPB) * C
        for par in range(2):
            @pl.when(lax.rem(blk, 2) == par)
            def _():
                xb = xblks[par]

                @pl.when(off == 0)
                def _():
                    pltpu.sync_copy(x_hbm.at[pl.ds(base + blk * BLK, BLK)], xb)
                    pltpu.sync_copy(seg_hbm.at[pl.ds(base + blk * BLK, BLK)],
                                    sb)

                pltpu.async_copy(word_hbm.at[xb.at[pl.ds(off, C)]], rows, gw)

        p0 = lax.rem(c * C, L)
        for u in range(C // 16):
            s16 = sb[pl.ds(off + u * 16, 16)]
            cidx[pl.ds(u * 16, 16)] = s16 * L + (iota + u * 16) + p0
        pltpu.async_copy(combo_hbm.at[cidx], addv, gc)

    def wait_in(c, buf):
        rows, addv, cidx, gw, gc, _ = buf
        blk = lax.div(c, CPB)
        off = lax.rem(c, CPB) * C
        for par in range(2):
            @pl.when(lax.rem(blk, 2) == par)
            def _():
                pltpu.make_async_copy(
                    word_hbm.at[xblks[par].at[pl.ds(off, C)]], rows, gw).wait()
        pltpu.make_async_copy(combo_hbm.at[cidx], addv, gc).wait()

    def accumulate(buf):
        rows, addv = buf[0], buf[1]

        def per_row(ci, _):
            for j in range(DSL):
                plsc.addupdate(rows.at[ci, pl.ds(j * 16, 16)],
                               addv[ci, pl.ds(j * 16, 16)])
            return 0

        lax.fori_loop(0, C, per_row, 0)

    def issue_out(c, buf):
        rows, wr = buf[0], buf[5]
        pltpu.async_copy(rows, out_hbm.at[pl.ds(base + c * C, C)], wr)

    def wait_out(buf):
        rows, wr = buf[0], buf[5]
        pltpu.make_async_copy(rows, out_hbm.at[pl.ds(base, C)], wr).wait()

    issue_in(0, bufs[0])

    def pair(i, carry):
        c0 = 2 * i
        for p in range(2):
            c = c0 + p
            cur, nxt = bufs[p], bufs[1 - p]

            @pl.when(c >= 1)
            def _():
                wait_out(nxt)

            @pl.when(c + 1 < N_CHUNK)
            def _():
                issue_in(c + 1, nxt)

            wait_in(c, cur)
            accumulate(cur)
            issue_out(c, cur)
        return carry

    lax.fori_loop(0, N_CHUNK // 2, pair, 0)
    wait_out(bufs[1])


@functools.partial(jax.jit, static_argnames=())
def _sc_embed(x_flat, seg_flat, word_table, combo):
    mesh = plsc.VectorSubcoreMesh(core_axis_name="c", subcore_axis_name="s",
                                  num_cores=NC, num_subcores=NS)
    f = pl.kernel(
        _sc_body,
        out_type=jax.ShapeDtypeStruct((N_TOK, D), jnp.float32),
        mesh=mesh,
        scratch_types=[
            pltpu.VMEM((C, D), jnp.float32),
            pltpu.VMEM((C, D), jnp.float32),
            pltpu.VMEM((C, D), jnp.float32),
            pltpu.VMEM((C, D), jnp.float32),
            pltpu.VMEM((C,), jnp.int32),
            pltpu.VMEM((C,), jnp.int32),
            pltpu.VMEM((BLK,), jnp.int32),
            pltpu.VMEM((BLK,), jnp.int32),
            pltpu.VMEM((BLK,), jnp.int32),
            pltpu.SemaphoreType.DMA,
            pltpu.SemaphoreType.DMA,
            pltpu.SemaphoreType.DMA,
            pltpu.SemaphoreType.DMA,
            pltpu.SemaphoreType.DMA,
            pltpu.SemaphoreType.DMA,
        ],
    )
    return f(x_flat, seg_flat, word_table, combo)


def _mask_body(x_ref, o_ref):
    o_ref[...] = x_ref[...] > 0


def _mask(x):
    return pl.pallas_call(
        _mask_body,
        out_shape=jax.ShapeDtypeStruct((B, L), jnp.bool_),
        grid=(8,),
        in_specs=[pl.BlockSpec((B // 8, L), lambda i: (i, 0))],
        out_specs=pl.BlockSpec((B // 8, L), lambda i: (i, 0)),
    )(x)


def kernel(x, seg, word_table, pos_table, seg_table):
    combo = (seg_table[:, None, :] + pos_table[None, :, :]).reshape(2 * L, D)
    out_flat = _sc_embed(x.reshape(N_TOK), seg.reshape(N_TOK), word_table,
                         combo)
    return out_flat.reshape(B, L, D), _mask(x)
